# baseline (speedup 1.0000x reference)
import functools

import jax
import jax.numpy as jnp
from jax import lax
from jax.experimental import pallas as pl
from jax.experimental.pallas import tpu as pltpu
from jax.experimental.pallas import tpu_sc as plsc

N = 10000
E = 320000
R = 500
D = 128
NP = 10240
RP = 512
EP = 327680
SLOPE = 0.2

NW = 32
EPW = EP // NW
CH = 128
NCH = EPW // CH
CHF = 64
NCHF = EPW // CHF
EPW0 = 8704
NCHF0 = EPW0 // CHF
NCHF1 = (2 * EPW - EPW0) // CHF
NPW = NP // 16


def _leaky(x):
    return jnp.maximum(x, SLOPE * x)


def _rel_kernel(rel_cat, w1, w2, wb, gcn, c_out, gr_out):
    ra = _leaky(jnp.dot(rel_cat[...], w1[...], preferred_element_type=jnp.float32))
    ra = jnp.dot(ra, w2[...], preferred_element_type=jnp.float32)
    c_out[...] = jnp.dot(ra, wb[...], preferred_element_type=jnp.float32)
    gr_out[...] = jnp.dot(ra, gcn[...], preferred_element_type=jnp.float32)


def _node_kernel(ent, wa, wb, gcn, a_out, b_out, gx_out):
    e = ent[...]
    a_out[...] = jnp.dot(e, wa[...], preferred_element_type=jnp.float32)
    b_out[...] = jnp.dot(e, wb[...], preferred_element_type=jnp.float32)
    gx_out[...] = jnp.dot(e, gcn[...], preferred_element_type=jnp.float32)


def _deginv_kernel(deg_p, deginv_out):
    deg = deg_p[0] + deg_p[1]
    deginv_out[...] = jnp.where(
        deg > 0, lax.rsqrt(jnp.maximum(deg, 1e-30)), 0.0)


def _invesum_kernel(esum_p, invesum_out):
    esum = esum_p[0] + esum_p[1]
    invesum_out[...] = jnp.where(
        esum > 0, 1.0 / jnp.where(esum > 0, esum, 1.0), 0.0)


def _final_kernel(acc_p, inv_esum, gx, gr_loop, gamma, beta, out):
    accsum = acc_p[0] + acc_p[1]
    msg_nb = accsum * inv_esum[...]
    h = (msg_nb + gx[...] - gr_loop[...]) * 0.5
    hn = h[:N]
    mean = jnp.mean(hn, axis=0, keepdims=True)
    var = jnp.mean(hn * hn, axis=0, keepdims=True) - mean * mean
    out[...] = jnp.tanh((hn - mean) * lax.rsqrt(var + 1e-5) * gamma[...] + beta[...])


def _tc_precompute(ent_emb, rel_emb, loop_rel, W1, W2, gcn_weight, w_att):
    wa = w_att[:D]
    wb = w_att[D:]
    rel_cat = jnp.concatenate(
        [rel_emb, loop_rel, jnp.zeros((RP - R - 1, D), jnp.float32)], axis=0)
    c_tab, gr_tab = pl.pallas_call(
        _rel_kernel,
        out_shape=[jax.ShapeDtypeStruct((RP, D), jnp.float32)] * 2,
    )(rel_cat, W1, W2, wb, gcn_weight)
    ent_p = jnp.concatenate([ent_emb, jnp.zeros((NP - N, D), jnp.float32)], axis=0)
    grid = (NP // 512,)
    bs = pl.BlockSpec((512, D), lambda i: (i, 0))
    ws = pl.BlockSpec((D, D), lambda i: (0, 0))
    a_tab, b_tab, gx_tab = pl.pallas_call(
        _node_kernel,
        grid=grid,
        in_specs=[bs, ws, ws, ws],
        out_specs=[bs, bs, bs],
        out_shape=[jax.ShapeDtypeStruct((NP, D), jnp.float32)] * 3,
    )(ent_p, wa, wb, gcn_weight)
    return a_tab, b_tab, gx_tab, c_tab, gr_tab


def _sc_deg_pass(src):
    mesh = plsc.VectorSubcoreMesh(core_axis_name="c", subcore_axis_name="s")

    @functools.partial(
        pl.kernel,
        out_type=jax.ShapeDtypeStruct((2, NP), jnp.float32),
        mesh=mesh,
        scratch_types=[
            pltpu.VMEM((CH,), jnp.int32),
            pltpu.VMEM((CH,), jnp.float32),
            pltpu.VMEM((NPW,), jnp.float32),
            pltpu.VMEM_SHARED((NP,), jnp.float32),
        ],
    )
    def sc_deg(src_hbm, deg_hbm, src_v, ones_v, st_v, deg_sh):
        cid = lax.axis_index("c")
        sid = lax.axis_index("s")
        wid = sid * 2 + cid
        base = wid * EPW
        zero16 = jnp.zeros((16,), jnp.float32)
        one16 = jnp.ones((16,), jnp.float32)

        for i in range(NPW // 16):
            st_v[pl.ds(i * 16, 16)] = zero16
        for i in range(CH // 16):
            ones_v[pl.ds(i * 16, 16)] = one16
        pltpu.sync_copy(st_v, deg_sh.at[pl.ds(sid * NPW, NPW)])
        plsc.subcore_barrier()

        def chunk_body(k, carry):
            cb = base + k * CH
            pltpu.sync_copy(src_hbm.at[pl.ds(cb, CH)], src_v)
            pltpu.sync_copy(ones_v, deg_sh.at[src_v], add=True)
            return carry

        lax.fori_loop(0, NCH, chunk_body, 0)
        plsc.subcore_barrier()
        pltpu.sync_copy(deg_sh.at[pl.ds(sid * NPW, NPW)], st_v)
        pltpu.sync_copy(st_v, deg_hbm.at[cid, pl.ds(sid * NPW, NPW)])

    return sc_deg(src)


def _sc_edge_fused(src, dst, et, a_tab, b_tab, c_tab, gx_tab, gr_tab,
                   deg_inv, a_vec):
    mesh = plsc.VectorSubcoreMesh(core_axis_name="c", subcore_axis_name="s")

    @functools.partial(
        pl.kernel,
        out_type=[jax.ShapeDtypeStruct((2, NP, D), jnp.float32),
                  jax.ShapeDtypeStruct((2, NP), jnp.float32)],
        mesh=mesh,
        scratch_types=[
            pltpu.VMEM((2, CHF), jnp.int32),
            pltpu.VMEM((2, CHF), jnp.int32),
            pltpu.VMEM((2, CHF), jnp.int32),
            pltpu.VMEM((CHF, D), jnp.float32),
            pltpu.VMEM((CHF, D), jnp.float32),
            pltpu.VMEM((CHF, D), jnp.float32),
            pltpu.VMEM((CHF, D), jnp.float32),
            pltpu.VMEM((CHF, D), jnp.float32),
            pltpu.VMEM((CHF,), jnp.float32),
            pltpu.VMEM((2, CHF), jnp.float32),
            pltpu.VMEM((D,), jnp.float32),
            pltpu.VMEM((8, D), jnp.float32),
            pltpu.VMEM((8, D), jnp.float32),
            pltpu.VMEM((NPW,), jnp.float32),
            pltpu.VMEM_SHARED((NP, D), jnp.float32),
            pltpu.VMEM_SHARED((NP,), jnp.float32),
            pltpu.SemaphoreType.DMA,
            pltpu.SemaphoreType.DMA,
            pltpu.SemaphoreType.DMA,
            pltpu.SemaphoreType.DMA,
        ],
    )
    def sc_f(src_hbm, dst_hbm, et_hbm, a_hbm, b_hbm, c_hbm, gx_hbm, gr_hbm,
             dinv_hbm, av_hbm, acc_hbm, esum_hbm,
             src_v, dst_v, et_v, a_v, b_v, c_v, gx_v, gr_v, dvg_v,
             exc_v, av_v, zr_v, st2_v, st_v, acc_sh, esum_sh,
             sem_i, sem_g, sem_s0, sem_s1):
        cid = lax.axis_index("c")
        sid = lax.axis_index("s")
        base = sid * 2 * EPW + cid * EPW0
        ncf = jnp.where(cid == 0, NCHF0, NCHF1)
        zero16 = jnp.zeros((16,), jnp.float32)
        sems_s = (sem_s0, sem_s1)

        pltpu.sync_copy(av_hbm, av_v)
        for r in range(8):
            for j in range(8):
                zr_v[r, pl.ds(j * 16, 16)] = zero16
        for i in range(NPW // 16):
            st_v[pl.ds(i * 16, 16)] = zero16
        for i in range(NPW // 8):
            pltpu.sync_copy(zr_v, acc_sh.at[pl.ds(sid * NPW + i * 8, 8), :])
        pltpu.sync_copy(st_v, esum_sh.at[pl.ds(sid * NPW, NPW)])
        plsc.subcore_barrier()

        def drain_scatter(p):
            pltpu.make_async_copy(gx_v, acc_sh.at[dst_v.at[p]],
                                  sems_s[p]).wait()
            pltpu.make_async_copy(exc_v.at[p], esum_sh.at[dst_v.at[p]],
                                  sems_s[p]).wait()

        def issue_idx(p, k):
            cb = base + k * CHF
            pltpu.async_copy(src_hbm.at[pl.ds(cb, CHF)], src_v.at[p], sem_i)
            pltpu.async_copy(dst_hbm.at[pl.ds(cb, CHF)], dst_v.at[p], sem_i)
            pltpu.async_copy(et_hbm.at[pl.ds(cb, CHF)], et_v.at[p], sem_i)

        def drain_idx(p, k):
            cb = base + k * CHF
            pltpu.make_async_copy(src_hbm.at[pl.ds(cb, CHF)], src_v.at[p],
                                  sem_i).wait()
            pltpu.make_async_copy(dst_hbm.at[pl.ds(cb, CHF)], dst_v.at[p],
                                  sem_i).wait()
            pltpu.make_async_copy(et_hbm.at[pl.ds(cb, CHF)], et_v.at[p],
                                  sem_i).wait()

        def do_chunk(p, k, have_prev, prefetch_cond):
            drain_idx(p, k)
            d1 = pltpu.async_copy(a_hbm.at[dst_v.at[p]], a_v, sem_g)
            d2 = pltpu.async_copy(b_hbm.at[src_v.at[p]], b_v, sem_g)
            d3 = pltpu.async_copy(c_hbm.at[et_v.at[p]], c_v, sem_g)
            d5 = pltpu.async_copy(gr_hbm.at[et_v.at[p]], gr_v, sem_g)
            d6 = pltpu.async_copy(dinv_hbm.at[src_v.at[p]], dvg_v, sem_g)
            if have_prev is None:
                drain_scatter(1 - p)
            else:
                @pl.when(have_prev)
                def _():
                    drain_scatter(1 - p)
            d4 = pltpu.async_copy(gx_hbm.at[src_v.at[p]], gx_v, sem_g)

            @pl.when(prefetch_cond)
            def _():
                issue_idx(1 - p, k + 1)

            d1.wait()
            d2.wait()
            d3.wait()
            d4.wait()
            d5.wait()
            d6.wait()

            def group_body(g, carry):
                ajs2 = tuple(av_v[pl.ds(j * 16, 16)] for j in range(8))
                io16 = lax.iota(jnp.int32, 16)
                sv = zero16
                for e2 in range(16):
                    ed = g * 16 + e2
                    acc = zero16
                    for j in range(8):
                        va = a_v[ed, pl.ds(j * 16, 16)]
                        vb = b_v[ed, pl.ds(j * 16, 16)]
                        vc = c_v[ed, pl.ds(j * 16, 16)]
                        v = va + vb - vc
                        acc = acc + jnp.maximum(v, SLOPE * v) * ajs2[j]
                    terms = [acc[l] for l in range(16)]
                    while len(terms) > 1:
                        terms = [terms[i2] + terms[i2 + 1]
                                 for i2 in range(0, len(terms), 2)]
                    sv = jnp.where(io16 == e2,
                                   jnp.broadcast_to(terms[0], (16,)), sv)
                exv = jnp.exp(sv)
                exc_v[p, pl.ds(g * 16, 16)] = exv
                coefv = exv * dvg_v[pl.ds(g * 16, 16)]
                for e2 in range(16):
                    ed = g * 16 + e2
                    ce = jnp.broadcast_to(coefv[e2], (16,))
                    for j in range(8):
                        gx = gx_v[ed, pl.ds(j * 16, 16)]
                        gr = gr_v[ed, pl.ds(j * 16, 16)]
                        gx_v[ed, pl.ds(j * 16, 16)] = ce * (gx - gr)
                return carry

            lax.fori_loop(0, CHF // 16, group_body, 0)
            pltpu.async_copy(gx_v, acc_sh.at[dst_v.at[p]], sems_s[p],
                             add=True)
            pltpu.async_copy(exc_v.at[p], esum_sh.at[dst_v.at[p]], sems_s[p],
                             add=True)

        issue_idx(0, 0)

        def pipe_body(i, carry):
            k0 = 2 * i
            do_chunk(0, k0, i > 0, k0 + 1 < ncf)
            do_chunk(1, k0 + 1, None, k0 + 2 < ncf)
            return carry

        lax.fori_loop(0, ncf // 2, pipe_body, 0)
        drain_scatter(1)
        plsc.subcore_barrier()
        for i in range(NPW // 8):
            rows = pl.ds(sid * NPW + i * 8, 8)
            pltpu.sync_copy(acc_sh.at[rows, :], st2_v)
            pltpu.sync_copy(st2_v, acc_hbm.at[cid, rows, :])
        pltpu.sync_copy(esum_sh.at[pl.ds(sid * NPW, NPW)], st_v)
        pltpu.sync_copy(st_v, esum_hbm.at[cid, pl.ds(sid * NPW, NPW)])

    return sc_f(src, dst, et, a_tab, b_tab, c_tab, gx_tab, gr_tab,
                deg_inv, a_vec)


def kernel(ent_emb, rel_emb, edge_index, edge_type, W1, W2, gcn_weight,
           loop_rel, w_att, a_att, bn_gamma, bn_beta):
    a_tab, b_tab, gx_tab, c_tab, gr_tab = _tc_precompute(
        ent_emb, rel_emb, loop_rel, W1, W2, gcn_weight, w_att)

    src = jnp.concatenate([edge_index[0], jnp.full((EP - E,), N, jnp.int32)])
    dst = jnp.concatenate([edge_index[1], jnp.full((EP - E,), N, jnp.int32)])
    et = jnp.concatenate([edge_type, jnp.full((EP - E,), R + 1, jnp.int32)])

    deg_p = _sc_deg_pass(src)

    deg_inv = pl.pallas_call(
        _deginv_kernel,
        out_shape=jax.ShapeDtypeStruct((NP // D, D), jnp.float32),
    )(deg_p.reshape(2, NP // D, D))

    acc_p, esum_p = _sc_edge_fused(src, dst, et, a_tab, b_tab, c_tab,
                                   gx_tab, gr_tab, deg_inv.reshape(NP),
                                   a_att[:, 0])

    inv_esum = pl.pallas_call(
        _invesum_kernel,
        out_shape=jax.ShapeDtypeStruct((NP // D, D), jnp.float32),
    )(esum_p.reshape(2, NP // D, D))

    return pl.pallas_call(
        _final_kernel,
        out_shape=jax.ShapeDtypeStruct((N, D), jnp.float32),
    )(acc_p, inv_esum.reshape(NP, 1), gx_tab, gr_tab[R].reshape(1, D),
      bn_gamma.reshape(1, D), bn_beta.reshape(1, D))


# speedup vs baseline: 1.1834x; 1.1834x over previous
import functools

import jax
import jax.numpy as jnp
from jax import lax
from jax.experimental import pallas as pl
from jax.experimental.pallas import tpu as pltpu
from jax.experimental.pallas import tpu_sc as plsc

N = 10000
E = 320000
R = 500
D = 128
NP = 10240
RP = 512
EP = 327680
SLOPE = 0.2

NW = 32
EPW = EP // NW
CH = 128
NCH = EPW // CH
CHF = 64
NCHF = EPW // CHF
EPW0 = 11776
NCHF0 = EPW0 // CHF
NCHF1 = (2 * EPW - EPW0) // CHF
NPW = NP // 16


def _leaky(x):
    return jnp.maximum(x, SLOPE * x)


def _rel_kernel(rel_cat, w1, w2, wb, gcn, c_out, gr_out):
    ra = _leaky(jnp.dot(rel_cat[...], w1[...], preferred_element_type=jnp.float32))
    ra = jnp.dot(ra, w2[...], preferred_element_type=jnp.float32)
    c_out[...] = jnp.dot(ra, wb[...], preferred_element_type=jnp.float32)
    gr_out[...] = jnp.dot(ra, gcn[...], preferred_element_type=jnp.float32)


def _node_kernel(ent, wa, wb, gcn, a_out, b_out, gx_out):
    e = ent[...]
    a_out[...] = jnp.dot(e, wa[...], preferred_element_type=jnp.float32)
    b_out[...] = jnp.dot(e, wb[...], preferred_element_type=jnp.float32)
    gx_out[...] = jnp.dot(e, gcn[...], preferred_element_type=jnp.float32)


def _deginv_kernel(deg_p, deginv_out):
    deg = deg_p[0] + deg_p[1]
    deginv_out[...] = jnp.where(
        deg > 0, lax.rsqrt(jnp.maximum(deg, 1e-30)), 0.0)


def _invesum_kernel(esum_p, invesum_out):
    esum = esum_p[0] + esum_p[1]
    invesum_out[...] = jnp.where(
        esum > 0, 1.0 / jnp.where(esum > 0, esum, 1.0), 0.0)


def _final_kernel(acc_p, inv_esum, gx, gr_loop, gamma, beta, out):
    accsum = acc_p[0] + acc_p[1]
    msg_nb = accsum * inv_esum[...]
    h = (msg_nb + gx[...] - gr_loop[...]) * 0.5
    hn = h[:N]
    mean = jnp.mean(hn, axis=0, keepdims=True)
    var = jnp.mean(hn * hn, axis=0, keepdims=True) - mean * mean
    out[...] = jnp.tanh((hn - mean) * lax.rsqrt(var + 1e-5) * gamma[...] + beta[...])


def _tc_precompute(ent_emb, rel_emb, loop_rel, W1, W2, gcn_weight, w_att):
    wa = w_att[:D]
    wb = w_att[D:]
    rel_cat = jnp.concatenate(
        [rel_emb, loop_rel, jnp.zeros((RP - R - 1, D), jnp.float32)], axis=0)
    c_tab, gr_tab = pl.pallas_call(
        _rel_kernel,
        out_shape=[jax.ShapeDtypeStruct((RP, D), jnp.float32)] * 2,
    )(rel_cat, W1, W2, wb, gcn_weight)
    ent_p = jnp.concatenate([ent_emb, jnp.zeros((NP - N, D), jnp.float32)], axis=0)
    grid = (NP // 512,)
    bs = pl.BlockSpec((512, D), lambda i: (i, 0))
    ws = pl.BlockSpec((D, D), lambda i: (0, 0))
    a_tab, b_tab, gx_tab = pl.pallas_call(
        _node_kernel,
        grid=grid,
        in_specs=[bs, ws, ws, ws],
        out_specs=[bs, bs, bs],
        out_shape=[jax.ShapeDtypeStruct((NP, D), jnp.float32)] * 3,
    )(ent_p, wa, wb, gcn_weight)
    return a_tab, b_tab, gx_tab, c_tab, gr_tab


def _sc_deg_pass(src):
    mesh = plsc.VectorSubcoreMesh(core_axis_name="c", subcore_axis_name="s")

    @functools.partial(
        pl.kernel,
        out_type=jax.ShapeDtypeStruct((2, NP), jnp.float32),
        mesh=mesh,
        scratch_types=[
            pltpu.VMEM((CH,), jnp.int32),
            pltpu.VMEM((CH,), jnp.float32),
            pltpu.VMEM((NPW,), jnp.float32),
            pltpu.VMEM_SHARED((NP,), jnp.float32),
        ],
    )
    def sc_deg(src_hbm, deg_hbm, src_v, ones_v, st_v, deg_sh):
        cid = lax.axis_index("c")
        sid = lax.axis_index("s")
        wid = sid * 2 + cid
        base = wid * EPW
        zero16 = jnp.zeros((16,), jnp.float32)
        one16 = jnp.ones((16,), jnp.float32)

        for i in range(NPW // 16):
            st_v[pl.ds(i * 16, 16)] = zero16
        for i in range(CH // 16):
            ones_v[pl.ds(i * 16, 16)] = one16
        pltpu.sync_copy(st_v, deg_sh.at[pl.ds(sid * NPW, NPW)])
        plsc.subcore_barrier()

        def chunk_body(k, carry):
            cb = base + k * CH
            pltpu.sync_copy(src_hbm.at[pl.ds(cb, CH)], src_v)
            pltpu.sync_copy(ones_v, deg_sh.at[src_v], add=True)
            return carry

        lax.fori_loop(0, NCH, chunk_body, 0)
        plsc.subcore_barrier()
        pltpu.sync_copy(deg_sh.at[pl.ds(sid * NPW, NPW)], st_v)
        pltpu.sync_copy(st_v, deg_hbm.at[cid, pl.ds(sid * NPW, NPW)])

    return sc_deg(src)


def _sc_edge_fused(src, dst, et, a_tab, b_tab, c_tab, gx_tab, gr_tab,
                   deg_inv, a_vec):
    mesh = plsc.VectorSubcoreMesh(core_axis_name="c", subcore_axis_name="s")

    @functools.partial(
        pl.kernel,
        out_type=[jax.ShapeDtypeStruct((2, NP, D), jnp.float32),
                  jax.ShapeDtypeStruct((2, NP), jnp.float32)],
        mesh=mesh,
        scratch_types=[
            pltpu.VMEM((2, CHF), jnp.int32),
            pltpu.VMEM((2, CHF), jnp.int32),
            pltpu.VMEM((2, CHF), jnp.int32),
            pltpu.VMEM((CHF, D), jnp.float32),
            pltpu.VMEM((CHF, D), jnp.float32),
            pltpu.VMEM((CHF, D), jnp.float32),
            pltpu.VMEM((CHF, D), jnp.float32),
            pltpu.VMEM((CHF, D), jnp.float32),
            pltpu.VMEM((CHF,), jnp.float32),
            pltpu.VMEM((2, CHF), jnp.float32),
            pltpu.VMEM((D,), jnp.float32),
            pltpu.VMEM((8, D), jnp.float32),
            pltpu.VMEM((8, D), jnp.float32),
            pltpu.VMEM((NPW,), jnp.float32),
            pltpu.VMEM_SHARED((NP, D), jnp.float32),
            pltpu.VMEM_SHARED((NP,), jnp.float32),
            pltpu.SemaphoreType.DMA,
            pltpu.SemaphoreType.DMA,
            pltpu.SemaphoreType.DMA,
            pltpu.SemaphoreType.DMA,
        ],
    )
    def sc_f(src_hbm, dst_hbm, et_hbm, a_hbm, b_hbm, c_hbm, gx_hbm, gr_hbm,
             dinv_hbm, av_hbm, acc_hbm, esum_hbm,
             src_v, dst_v, et_v, a_v, b_v, c_v, gx_v, gr_v, dvg_v,
             exc_v, av_v, zr_v, st2_v, st_v, acc_sh, esum_sh,
             sem_i, sem_g, sem_s0, sem_s1):
        cid = lax.axis_index("c")
        sid = lax.axis_index("s")
        base = sid * 2 * EPW + cid * EPW0
        ncf = jnp.where(cid == 0, NCHF0, NCHF1)
        zero16 = jnp.zeros((16,), jnp.float32)
        sems_s = (sem_s0, sem_s1)

        pltpu.sync_copy(av_hbm, av_v)
        for r in range(8):
            for j in range(8):
                zr_v[r, pl.ds(j * 16, 16)] = zero16
        for i in range(NPW // 16):
            st_v[pl.ds(i * 16, 16)] = zero16
        for i in range(NPW // 8):
            pltpu.sync_copy(zr_v, acc_sh.at[pl.ds(sid * NPW + i * 8, 8), :])
        pltpu.sync_copy(st_v, esum_sh.at[pl.ds(sid * NPW, NPW)])
        plsc.subcore_barrier()

        def drain_scatter(p):
            pltpu.make_async_copy(gx_v, acc_sh.at[dst_v.at[p]],
                                  sems_s[p]).wait()
            pltpu.make_async_copy(exc_v.at[p], esum_sh.at[dst_v.at[p]],
                                  sems_s[p]).wait()

        def issue_idx(p, k):
            cb = base + k * CHF
            pltpu.async_copy(src_hbm.at[pl.ds(cb, CHF)], src_v.at[p], sem_i)
            pltpu.async_copy(dst_hbm.at[pl.ds(cb, CHF)], dst_v.at[p], sem_i)
            pltpu.async_copy(et_hbm.at[pl.ds(cb, CHF)], et_v.at[p], sem_i)

        def drain_idx(p, k):
            cb = base + k * CHF
            pltpu.make_async_copy(src_hbm.at[pl.ds(cb, CHF)], src_v.at[p],
                                  sem_i).wait()
            pltpu.make_async_copy(dst_hbm.at[pl.ds(cb, CHF)], dst_v.at[p],
                                  sem_i).wait()
            pltpu.make_async_copy(et_hbm.at[pl.ds(cb, CHF)], et_v.at[p],
                                  sem_i).wait()

        def do_chunk(p, k, have_prev, prefetch_cond):
            drain_idx(p, k)
            d1 = pltpu.async_copy(a_hbm.at[dst_v.at[p]], a_v, sem_g)
            d2 = pltpu.async_copy(b_hbm.at[src_v.at[p]], b_v, sem_g)
            d3 = pltpu.async_copy(c_hbm.at[et_v.at[p]], c_v, sem_g)
            d5 = pltpu.async_copy(gr_hbm.at[et_v.at[p]], gr_v, sem_g)
            d6 = pltpu.async_copy(dinv_hbm.at[src_v.at[p]], dvg_v, sem_g)
            if have_prev is None:
                drain_scatter(1 - p)
            else:
                @pl.when(have_prev)
                def _():
                    drain_scatter(1 - p)
            d4 = pltpu.async_copy(gx_hbm.at[src_v.at[p]], gx_v, sem_g)

            @pl.when(prefetch_cond)
            def _():
                issue_idx(1 - p, k + 1)

            d1.wait()
            d2.wait()
            d3.wait()
            d4.wait()
            d5.wait()
            d6.wait()

            def group_body(g, carry):
                ajs2 = tuple(av_v[pl.ds(j * 16, 16)] for j in range(8))
                io16 = lax.iota(jnp.int32, 16)
                sv = zero16
                for e2 in range(16):
                    ed = g * 16 + e2
                    acc = zero16
                    for j in range(8):
                        va = a_v[ed, pl.ds(j * 16, 16)]
                        vb = b_v[ed, pl.ds(j * 16, 16)]
                        vc = c_v[ed, pl.ds(j * 16, 16)]
                        v = va + vb - vc
                        acc = acc + jnp.maximum(v, SLOPE * v) * ajs2[j]
                    terms = [acc[l] for l in range(16)]
                    while len(terms) > 1:
                        terms = [terms[i2] + terms[i2 + 1]
                                 for i2 in range(0, len(terms), 2)]
                    sv = jnp.where(io16 == e2,
                                   jnp.broadcast_to(terms[0], (16,)), sv)
                exv = jnp.exp(sv)
                exc_v[p, pl.ds(g * 16, 16)] = exv
                coefv = exv * dvg_v[pl.ds(g * 16, 16)]
                for e2 in range(16):
                    ed = g * 16 + e2
                    ce = jnp.broadcast_to(coefv[e2], (16,))
                    for j in range(8):
                        gx = gx_v[ed, pl.ds(j * 16, 16)]
                        gr = gr_v[ed, pl.ds(j * 16, 16)]
                        gx_v[ed, pl.ds(j * 16, 16)] = ce * (gx - gr)
                return carry

            lax.fori_loop(0, CHF // 16, group_body, 0)
            pltpu.async_copy(gx_v, acc_sh.at[dst_v.at[p]], sems_s[p],
                             add=True)
            pltpu.async_copy(exc_v.at[p], esum_sh.at[dst_v.at[p]], sems_s[p],
                             add=True)

        issue_idx(0, 0)

        def pipe_body(i, carry):
            k0 = 2 * i
            do_chunk(0, k0, i > 0, k0 + 1 < ncf)
            do_chunk(1, k0 + 1, None, k0 + 2 < ncf)
            return carry

        lax.fori_loop(0, ncf // 2, pipe_body, 0)
        drain_scatter(1)
        plsc.subcore_barrier()
        for i in range(NPW // 8):
            rows = pl.ds(sid * NPW + i * 8, 8)
            pltpu.sync_copy(acc_sh.at[rows, :], st2_v)
            pltpu.sync_copy(st2_v, acc_hbm.at[cid, rows, :])
        pltpu.sync_copy(esum_sh.at[pl.ds(sid * NPW, NPW)], st_v)
        pltpu.sync_copy(st_v, esum_hbm.at[cid, pl.ds(sid * NPW, NPW)])

    return sc_f(src, dst, et, a_tab, b_tab, c_tab, gx_tab, gr_tab,
                deg_inv, a_vec)


def kernel(ent_emb, rel_emb, edge_index, edge_type, W1, W2, gcn_weight,
           loop_rel, w_att, a_att, bn_gamma, bn_beta):
    a_tab, b_tab, gx_tab, c_tab, gr_tab = _tc_precompute(
        ent_emb, rel_emb, loop_rel, W1, W2, gcn_weight, w_att)

    src = jnp.concatenate([edge_index[0], jnp.full((EP - E,), N, jnp.int32)])
    dst = jnp.concatenate([edge_index[1], jnp.full((EP - E,), N, jnp.int32)])
    et = jnp.concatenate([edge_type, jnp.full((EP - E,), R + 1, jnp.int32)])

    deg_p = _sc_deg_pass(src)

    deg_inv = pl.pallas_call(
        _deginv_kernel,
        out_shape=jax.ShapeDtypeStruct((NP // D, D), jnp.float32),
    )(deg_p.reshape(2, NP // D, D))

    acc_p, esum_p = _sc_edge_fused(src, dst, et, a_tab, b_tab, c_tab,
                                   gx_tab, gr_tab, deg_inv.reshape(NP),
                                   a_att[:, 0])

    inv_esum = pl.pallas_call(
        _invesum_kernel,
        out_shape=jax.ShapeDtypeStruct((NP // D, D), jnp.float32),
    )(esum_p.reshape(2, NP // D, D))

    return pl.pallas_call(
        _final_kernel,
        out_shape=jax.ShapeDtypeStruct((N, D), jnp.float32),
    )(acc_p, inv_esum.reshape(NP, 1), gx_tab, gr_tab[R].reshape(1, D),
      bn_gamma.reshape(1, D), bn_beta.reshape(1, D))


# speedup vs baseline: 1.2213x; 1.0320x over previous
import functools

import jax
import jax.numpy as jnp
from jax import lax
from jax.experimental import pallas as pl
from jax.experimental.pallas import tpu as pltpu
from jax.experimental.pallas import tpu_sc as plsc

N = 10000
E = 320000
R = 500
D = 128
NP = 10240
RP = 512
EP = 327680
SLOPE = 0.2

NW = 32
EPW = EP // NW
CH = 128
NCH = EPW // CH
CHF = 64
NCHF = EPW // CHF
EPW0 = 12288
NCHF0 = EPW0 // CHF
NCHF1 = (2 * EPW - EPW0) // CHF
NPW = NP // 16


def _leaky(x):
    return jnp.maximum(x, SLOPE * x)


def _rel_kernel(rel_cat, w1, w2, wb, gcn, c_out, gr_out):
    ra = _leaky(jnp.dot(rel_cat[...], w1[...], preferred_element_type=jnp.float32))
    ra = jnp.dot(ra, w2[...], preferred_element_type=jnp.float32)
    c_out[...] = jnp.dot(ra, wb[...], preferred_element_type=jnp.float32)
    gr_out[...] = jnp.dot(ra, gcn[...], preferred_element_type=jnp.float32)


def _node_kernel(ent, wa, wb, gcn, a_out, b_out, gx_out):
    e = ent[...]
    a_out[...] = jnp.dot(e, wa[...], preferred_element_type=jnp.float32)
    b_out[...] = jnp.dot(e, wb[...], preferred_element_type=jnp.float32)
    gx_out[...] = jnp.dot(e, gcn[...], preferred_element_type=jnp.float32)


def _deginv_kernel(deg_p, deginv_out):
    deg = deg_p[0] + deg_p[1]
    deginv_out[...] = jnp.where(
        deg > 0, lax.rsqrt(jnp.maximum(deg, 1e-30)), 0.0)


def _invesum_kernel(esum_p, invesum_out):
    esum = esum_p[0] + esum_p[1]
    invesum_out[...] = jnp.where(
        esum > 0, 1.0 / jnp.where(esum > 0, esum, 1.0), 0.0)


def _final_kernel(acc_p, inv_esum, gx, gr_loop, gamma, beta, out):
    accsum = acc_p[0] + acc_p[1]
    msg_nb = accsum * inv_esum[...]
    h = (msg_nb + gx[...] - gr_loop[...]) * 0.5
    hn = h[:N]
    mean = jnp.mean(hn, axis=0, keepdims=True)
    var = jnp.mean(hn * hn, axis=0, keepdims=True) - mean * mean
    out[...] = jnp.tanh((hn - mean) * lax.rsqrt(var + 1e-5) * gamma[...] + beta[...])


def _tc_precompute(ent_emb, rel_emb, loop_rel, W1, W2, gcn_weight, w_att):
    wa = w_att[:D]
    wb = w_att[D:]
    rel_cat = jnp.concatenate(
        [rel_emb, loop_rel, jnp.zeros((RP - R - 1, D), jnp.float32)], axis=0)
    c_tab, gr_tab = pl.pallas_call(
        _rel_kernel,
        out_shape=[jax.ShapeDtypeStruct((RP, D), jnp.float32)] * 2,
    )(rel_cat, W1, W2, wb, gcn_weight)
    ent_p = jnp.concatenate([ent_emb, jnp.zeros((NP - N, D), jnp.float32)], axis=0)
    grid = (NP // 512,)
    bs = pl.BlockSpec((512, D), lambda i: (i, 0))
    ws = pl.BlockSpec((D, D), lambda i: (0, 0))
    a_tab, b_tab, gx_tab = pl.pallas_call(
        _node_kernel,
        grid=grid,
        in_specs=[bs, ws, ws, ws],
        out_specs=[bs, bs, bs],
        out_shape=[jax.ShapeDtypeStruct((NP, D), jnp.float32)] * 3,
    )(ent_p, wa, wb, gcn_weight)
    return a_tab, b_tab, gx_tab, c_tab, gr_tab


def _sc_deg_pass(src):
    mesh = plsc.VectorSubcoreMesh(core_axis_name="c", subcore_axis_name="s")

    @functools.partial(
        pl.kernel,
        out_type=jax.ShapeDtypeStruct((2, NP), jnp.float32),
        mesh=mesh,
        scratch_types=[
            pltpu.VMEM((CH,), jnp.int32),
            pltpu.VMEM((CH,), jnp.float32),
            pltpu.VMEM((NPW,), jnp.float32),
            pltpu.VMEM_SHARED((NP,), jnp.float32),
        ],
    )
    def sc_deg(src_hbm, deg_hbm, src_v, ones_v, st_v, deg_sh):
        cid = lax.axis_index("c")
        sid = lax.axis_index("s")
        wid = sid * 2 + cid
        base = wid * EPW
        zero16 = jnp.zeros((16,), jnp.float32)
        one16 = jnp.ones((16,), jnp.float32)

        for i in range(NPW // 16):
            st_v[pl.ds(i * 16, 16)] = zero16
        for i in range(CH // 16):
            ones_v[pl.ds(i * 16, 16)] = one16
        pltpu.sync_copy(st_v, deg_sh.at[pl.ds(sid * NPW, NPW)])
        plsc.subcore_barrier()

        def chunk_body(k, carry):
            cb = base + k * CH
            pltpu.sync_copy(src_hbm.at[pl.ds(cb, CH)], src_v)
            pltpu.sync_copy(ones_v, deg_sh.at[src_v], add=True)
            return carry

        lax.fori_loop(0, NCH, chunk_body, 0)
        plsc.subcore_barrier()
        pltpu.sync_copy(deg_sh.at[pl.ds(sid * NPW, NPW)], st_v)
        pltpu.sync_copy(st_v, deg_hbm.at[cid, pl.ds(sid * NPW, NPW)])

    return sc_deg(src)


def _sc_edge_fused(src, dst, et, a_tab, b_tab, c_tab, gx_tab, gr_tab,
                   deg_inv, a_vec):
    mesh = plsc.VectorSubcoreMesh(core_axis_name="c", subcore_axis_name="s")

    @functools.partial(
        pl.kernel,
        out_type=[jax.ShapeDtypeStruct((2, NP, D), jnp.float32),
                  jax.ShapeDtypeStruct((2, NP), jnp.float32)],
        mesh=mesh,
        scratch_types=[
            pltpu.VMEM((2, CHF), jnp.int32),
            pltpu.VMEM((2, CHF), jnp.int32),
            pltpu.VMEM((2, CHF), jnp.int32),
            pltpu.VMEM((CHF, D), jnp.float32),
            pltpu.VMEM((CHF, D), jnp.float32),
            pltpu.VMEM((CHF, D), jnp.float32),
            pltpu.VMEM((CHF, D), jnp.float32),
            pltpu.VMEM((CHF, D), jnp.float32),
            pltpu.VMEM((CHF,), jnp.float32),
            pltpu.VMEM((2, CHF), jnp.float32),
            pltpu.VMEM((D,), jnp.float32),
            pltpu.VMEM((8, D), jnp.float32),
            pltpu.VMEM((8, D), jnp.float32),
            pltpu.VMEM((NPW,), jnp.float32),
            pltpu.VMEM_SHARED((NP, D), jnp.float32),
            pltpu.VMEM_SHARED((NP,), jnp.float32),
            pltpu.SemaphoreType.DMA,
            pltpu.SemaphoreType.DMA,
            pltpu.SemaphoreType.DMA,
            pltpu.SemaphoreType.DMA,
        ],
    )
    def sc_f(src_hbm, dst_hbm, et_hbm, a_hbm, b_hbm, c_hbm, gx_hbm, gr_hbm,
             dinv_hbm, av_hbm, acc_hbm, esum_hbm,
             src_v, dst_v, et_v, a_v, b_v, c_v, gx_v, gr_v, dvg_v,
             exc_v, av_v, zr_v, st2_v, st_v, acc_sh, esum_sh,
             sem_i, sem_g, sem_s0, sem_s1):
        cid = lax.axis_index("c")
        sid = lax.axis_index("s")
        base = sid * 2 * EPW + cid * EPW0
        ncf = jnp.where(cid == 0, NCHF0, NCHF1)
        zero16 = jnp.zeros((16,), jnp.float32)
        sems_s = (sem_s0, sem_s1)

        pltpu.sync_copy(av_hbm, av_v)
        for r in range(8):
            for j in range(8):
                zr_v[r, pl.ds(j * 16, 16)] = zero16
        for i in range(NPW // 16):
            st_v[pl.ds(i * 16, 16)] = zero16
        for i in range(NPW // 8):
            pltpu.sync_copy(zr_v, acc_sh.at[pl.ds(sid * NPW + i * 8, 8), :])
        pltpu.sync_copy(st_v, esum_sh.at[pl.ds(sid * NPW, NPW)])
        plsc.subcore_barrier()

        def drain_scatter(p):
            pltpu.make_async_copy(gx_v, acc_sh.at[dst_v.at[p]],
                                  sems_s[p]).wait()
            pltpu.make_async_copy(exc_v.at[p], esum_sh.at[dst_v.at[p]],
                                  sems_s[p]).wait()

        def issue_idx(p, k):
            cb = base + k * CHF
            pltpu.async_copy(src_hbm.at[pl.ds(cb, CHF)], src_v.at[p], sem_i)
            pltpu.async_copy(dst_hbm.at[pl.ds(cb, CHF)], dst_v.at[p], sem_i)
            pltpu.async_copy(et_hbm.at[pl.ds(cb, CHF)], et_v.at[p], sem_i)

        def drain_idx(p, k):
            cb = base + k * CHF
            pltpu.make_async_copy(src_hbm.at[pl.ds(cb, CHF)], src_v.at[p],
                                  sem_i).wait()
            pltpu.make_async_copy(dst_hbm.at[pl.ds(cb, CHF)], dst_v.at[p],
                                  sem_i).wait()
            pltpu.make_async_copy(et_hbm.at[pl.ds(cb, CHF)], et_v.at[p],
                                  sem_i).wait()

        def do_chunk(p, k, have_prev, prefetch_cond):
            drain_idx(p, k)
            d1 = pltpu.async_copy(a_hbm.at[dst_v.at[p]], a_v, sem_g)
            d2 = pltpu.async_copy(b_hbm.at[src_v.at[p]], b_v, sem_g)
            d3 = pltpu.async_copy(c_hbm.at[et_v.at[p]], c_v, sem_g)
            d5 = pltpu.async_copy(gr_hbm.at[et_v.at[p]], gr_v, sem_g)
            d6 = pltpu.async_copy(dinv_hbm.at[src_v.at[p]], dvg_v, sem_g)
            if have_prev is None:
                drain_scatter(1 - p)
            else:
                @pl.when(have_prev)
                def _():
                    drain_scatter(1 - p)
            d4 = pltpu.async_copy(gx_hbm.at[src_v.at[p]], gx_v, sem_g)

            @pl.when(prefetch_cond)
            def _():
                issue_idx(1 - p, k + 1)

            d1.wait()
            d2.wait()
            d3.wait()
            d4.wait()
            d5.wait()
            d6.wait()

            def group_body(g, carry):
                ajs2 = tuple(av_v[pl.ds(j * 16, 16)] for j in range(8))
                io16 = lax.iota(jnp.int32, 16)
                sv = zero16
                for e2 in range(16):
                    ed = g * 16 + e2
                    acc = zero16
                    for j in range(8):
                        va = a_v[ed, pl.ds(j * 16, 16)]
                        vb = b_v[ed, pl.ds(j * 16, 16)]
                        vc = c_v[ed, pl.ds(j * 16, 16)]
                        v = va + vb - vc
                        acc = acc + jnp.maximum(v, SLOPE * v) * ajs2[j]
                    terms = [acc[l] for l in range(16)]
                    while len(terms) > 1:
                        terms = [terms[i2] + terms[i2 + 1]
                                 for i2 in range(0, len(terms), 2)]
                    sv = jnp.where(io16 == e2,
                                   jnp.broadcast_to(terms[0], (16,)), sv)
                exv = jnp.exp(sv)
                exc_v[p, pl.ds(g * 16, 16)] = exv
                coefv = exv * dvg_v[pl.ds(g * 16, 16)]
                for e2 in range(16):
                    ed = g * 16 + e2
                    ce = jnp.broadcast_to(coefv[e2], (16,))
                    for j in range(8):
                        gx = gx_v[ed, pl.ds(j * 16, 16)]
                        gr = gr_v[ed, pl.ds(j * 16, 16)]
                        gx_v[ed, pl.ds(j * 16, 16)] = ce * (gx - gr)
                return carry

            lax.fori_loop(0, CHF // 16, group_body, 0)
            pltpu.async_copy(gx_v, acc_sh.at[dst_v.at[p]], sems_s[p],
                             add=True)
            pltpu.async_copy(exc_v.at[p], esum_sh.at[dst_v.at[p]], sems_s[p],
                             add=True)

        issue_idx(0, 0)

        def pipe_body(i, carry):
            k0 = 2 * i
            do_chunk(0, k0, i > 0, k0 + 1 < ncf)
            do_chunk(1, k0 + 1, None, k0 + 2 < ncf)
            return carry

        lax.fori_loop(0, ncf // 2, pipe_body, 0)
        drain_scatter(1)
        plsc.subcore_barrier()
        for i in range(NPW // 8):
            rows = pl.ds(sid * NPW + i * 8, 8)
            pltpu.sync_copy(acc_sh.at[rows, :], st2_v)
            pltpu.sync_copy(st2_v, acc_hbm.at[cid, rows, :])
        pltpu.sync_copy(esum_sh.at[pl.ds(sid * NPW, NPW)], st_v)
        pltpu.sync_copy(st_v, esum_hbm.at[cid, pl.ds(sid * NPW, NPW)])

    return sc_f(src, dst, et, a_tab, b_tab, c_tab, gx_tab, gr_tab,
                deg_inv, a_vec)


def kernel(ent_emb, rel_emb, edge_index, edge_type, W1, W2, gcn_weight,
           loop_rel, w_att, a_att, bn_gamma, bn_beta):
    a_tab, b_tab, gx_tab, c_tab, gr_tab = _tc_precompute(
        ent_emb, rel_emb, loop_rel, W1, W2, gcn_weight, w_att)

    src = jnp.concatenate([edge_index[0], jnp.full((EP - E,), N, jnp.int32)])
    dst = jnp.concatenate([edge_index[1], jnp.full((EP - E,), N, jnp.int32)])
    et = jnp.concatenate([edge_type, jnp.full((EP - E,), R + 1, jnp.int32)])

    deg_p = _sc_deg_pass(src)

    deg_inv = pl.pallas_call(
        _deginv_kernel,
        out_shape=jax.ShapeDtypeStruct((NP // D, D), jnp.float32),
    )(deg_p.reshape(2, NP // D, D))

    acc_p, esum_p = _sc_edge_fused(src, dst, et, a_tab, b_tab, c_tab,
                                   gx_tab, gr_tab, deg_inv.reshape(NP),
                                   a_att[:, 0])

    inv_esum = pl.pallas_call(
        _invesum_kernel,
        out_shape=jax.ShapeDtypeStruct((NP // D, D), jnp.float32),
    )(esum_p.reshape(2, NP // D, D))

    return pl.pallas_call(
        _final_kernel,
        out_shape=jax.ShapeDtypeStruct((N, D), jnp.float32),
    )(acc_p, inv_esum.reshape(NP, 1), gx_tab, gr_tab[R].reshape(1, D),
      bn_gamma.reshape(1, D), bn_beta.reshape(1, D))
